# baseline (device time: 4590 ns/iter reference)
import jax
import jax.numpy as jnp
from jax import lax
from jax.experimental import pallas as pl
from jax.experimental.pallas import tpu as pltpu

N_CHUNKS = 4


def kernel(x):
    m, n = x.shape
    m_chunk = m // N_CHUNKS

    def body(x_ref, out_ref, acc_ref):
        j = pl.program_id(0)
        part = jnp.sum(x_ref[:, :], axis=0, keepdims=True)

        @pl.when(j == 0)
        def _():
            acc_ref[:, :] = part

        @pl.when((j > 0) & (j < N_CHUNKS - 1))
        def _():
            acc_ref[:, :] = acc_ref[:, :] + part

        @pl.when(j == N_CHUNKS - 1)
        def _():
            out_ref[:, :] = acc_ref[:, :] + part

    return pl.pallas_call(
        body,
        grid=(N_CHUNKS,),
        out_shape=jax.ShapeDtypeStruct((1, n), x.dtype),
        in_specs=[pl.BlockSpec((m_chunk, n), lambda j: (j, 0))],
        out_specs=pl.BlockSpec((1, n), lambda j: (0, 0)),
        scratch_shapes=[pltpu.VMEM((1, n), x.dtype)],
    )(x)
